# baseline (speedup 1.0000x reference)
import functools

import jax
import jax.numpy as jnp
from jax import lax
from jax.experimental import pallas as pl
from jax.experimental.pallas import tpu as pltpu
from jax.experimental.pallas import tpu_sc as plsc

NUM_E = 100000
D = 300
DP = 384
BATCH = 4096
HIST = 50
NC, NS = 2, 16
NW = NC * NS
HP = 56
BP = BATCH * HP
CHUNK = 64
NBUF = 4
CPW = BP // NW // CHUNK

_mesh = plsc.VectorSubcoreMesh(core_axis_name="c", subcore_axis_name="s")


@functools.partial(
    pl.kernel,
    mesh=_mesh,
    out_type=jax.ShapeDtypeStruct((BP, DP), jnp.float32),
    scratch_types=[
        pltpu.VMEM((CPW, CHUNK), jnp.int32),
    ]
    + [pltpu.VMEM((CHUNK, DP), jnp.float32) for _ in range(NBUF)]
    + [pltpu.SemaphoreType.DMA for _ in range(2 * NBUF)],
    compiler_params=pltpu.CompilerParams(use_tc_tiling_on_sc=True),
)
def _gather(x_hbm, w_hbm, out_hbm, idx_v, *bufs_and_sems):
    rows = bufs_and_sems[:NBUF]
    sg = bufs_and_sems[NBUF : 2 * NBUF]
    sw = bufs_and_sems[2 * NBUF :]
    wid = lax.axis_index("s") * NC + lax.axis_index("c")
    pltpu.sync_copy(x_hbm.at[wid], idx_v)
    base = wid * (CPW * CHUNK)

    def body(i, carry):
        for b in range(NBUF):
            j = i * NBUF + b

            @pl.when(i > 0)
            def _():
                pltpu.make_async_copy(
                    rows[b], out_hbm.at[pl.ds(base, CHUNK)], sw[b]
                ).wait()

            pltpu.async_copy(w_hbm.at[idx_v.at[j]], rows[b], sg[b]).wait()
            pltpu.async_copy(
                rows[b], out_hbm.at[pl.ds(base + j * CHUNK, CHUNK)], sw[b]
            )
        return carry

    lax.fori_loop(0, CPW // NBUF, body, 0)
    for b in range(NBUF):
        pltpu.make_async_copy(
            rows[b], out_hbm.at[pl.ds(base, CHUNK)], sw[b]
        ).wait()


_TR_BLOCK = 2048


def _transpose_block(wt_ref, wp_ref):
    wp_ref[...] = jnp.transpose(wt_ref[...], (1, 0))


_transpose = pl.pallas_call(
    _transpose_block,
    grid=(pl.cdiv(NUM_E, _TR_BLOCK),),
    in_specs=[pl.BlockSpec((DP, _TR_BLOCK), lambda i: (0, i))],
    out_specs=pl.BlockSpec((_TR_BLOCK, DP), lambda i: (i, 0)),
    out_shape=jax.ShapeDtypeStruct((NUM_E, DP), jnp.float32),
)


def kernel(x, weight):
    xp = jnp.pad(x, ((0, 0), (0, HP - HIST)), mode="edge")
    xr = xp.reshape(NW, CPW, CHUNK)
    wp = _transpose(lax.transpose(weight, (1, 0)))
    out = _gather(xr, wp)
    return out.reshape(BATCH, HP, DP)[:, :HIST, :D]


# speedup vs baseline: 1.0857x; 1.0857x over previous
import functools

import jax
import jax.numpy as jnp
from jax import lax
from jax.experimental import pallas as pl
from jax.experimental.pallas import tpu as pltpu
from jax.experimental.pallas import tpu_sc as plsc

NUM_E = 100000
D = 300
DP = 384
BATCH = 4096
HIST = 50
NC, NS = 2, 16
NW = NC * NS
HP = 56
BP = BATCH * HP
CHUNK = 128
NBUF = 2
CPW = BP // NW // CHUNK

_mesh = plsc.VectorSubcoreMesh(core_axis_name="c", subcore_axis_name="s")


@functools.partial(
    pl.kernel,
    mesh=_mesh,
    out_type=jax.ShapeDtypeStruct((BP, DP), jnp.float32),
    scratch_types=[
        pltpu.VMEM((CPW, CHUNK), jnp.int32),
    ]
    + [pltpu.VMEM((CHUNK, DP), jnp.float32) for _ in range(NBUF)]
    + [pltpu.SemaphoreType.DMA for _ in range(2 * NBUF)],
    compiler_params=pltpu.CompilerParams(use_tc_tiling_on_sc=True),
)
def _gather(x_hbm, w_hbm, out_hbm, idx_v, *bufs_and_sems):
    rows = bufs_and_sems[:NBUF]
    sg = bufs_and_sems[NBUF : 2 * NBUF]
    sw = bufs_and_sems[2 * NBUF :]
    wid = lax.axis_index("s") * NC + lax.axis_index("c")
    pltpu.sync_copy(x_hbm.at[wid], idx_v)
    base = wid * (CPW * CHUNK)

    def body(i, carry):
        for b in range(NBUF):
            j = i * NBUF + b

            @pl.when(i > 0)
            def _():
                pltpu.make_async_copy(
                    rows[b], out_hbm.at[pl.ds(base, CHUNK)], sw[b]
                ).wait()

            pltpu.async_copy(w_hbm.at[idx_v.at[j]], rows[b], sg[b]).wait()
            pltpu.async_copy(
                rows[b], out_hbm.at[pl.ds(base + j * CHUNK, CHUNK)], sw[b]
            )
        return carry

    lax.fori_loop(0, CPW // NBUF, body, 0)
    for b in range(NBUF):
        pltpu.make_async_copy(
            rows[b], out_hbm.at[pl.ds(base, CHUNK)], sw[b]
        ).wait()


_TR_BLOCK = 2048


def _transpose_block(wt_ref, wp_ref):
    wp_ref[...] = jnp.transpose(wt_ref[...], (1, 0))


_transpose = pl.pallas_call(
    _transpose_block,
    grid=(pl.cdiv(NUM_E, _TR_BLOCK),),
    in_specs=[pl.BlockSpec((DP, _TR_BLOCK), lambda i: (0, i))],
    out_specs=pl.BlockSpec((_TR_BLOCK, DP), lambda i: (i, 0)),
    out_shape=jax.ShapeDtypeStruct((NUM_E, DP), jnp.float32),
)


def kernel(x, weight):
    xp = jnp.pad(x, ((0, 0), (0, HP - HIST)), mode="edge")
    xr = xp.reshape(NW, CPW, CHUNK)
    wp = _transpose(lax.transpose(weight, (1, 0)))
    out = _gather(xr, wp)
    return out.reshape(BATCH, HP, DP)[:, :HIST, :D]


# speedup vs baseline: 1.0875x; 1.0017x over previous
import functools

import jax
import jax.numpy as jnp
from jax import lax
from jax.experimental import pallas as pl
from jax.experimental.pallas import tpu as pltpu
from jax.experimental.pallas import tpu_sc as plsc

NUM_E = 100000
D = 300
DP = 384
BATCH = 4096
HIST = 50
NC, NS = 2, 16
NW = NC * NS
HP = 56
BP = BATCH * HP
CHUNK = 128
NBUF = 2
CPW = BP // NW // CHUNK

_mesh = plsc.VectorSubcoreMesh(core_axis_name="c", subcore_axis_name="s")


@functools.partial(
    pl.kernel,
    mesh=_mesh,
    out_type=jax.ShapeDtypeStruct((BP, DP), jnp.float32),
    scratch_types=[
        pltpu.VMEM((CPW, CHUNK), jnp.int32),
    ]
    + [pltpu.VMEM((CHUNK, DP), jnp.float32) for _ in range(NBUF)]
    + [pltpu.SemaphoreType.DMA for _ in range(2 * NBUF)],
    compiler_params=pltpu.CompilerParams(use_tc_tiling_on_sc=True),
)
def _gather(x_hbm, w_hbm, out_hbm, idx_v, *bufs_and_sems):
    rows = bufs_and_sems[:NBUF]
    sg = bufs_and_sems[NBUF : 2 * NBUF]
    sw = bufs_and_sems[2 * NBUF :]
    wid = lax.axis_index("s") * NC + lax.axis_index("c")
    pltpu.sync_copy(x_hbm.at[wid], idx_v)
    base = wid * (CPW * CHUNK)

    def body(i, carry):
        for b in range(NBUF):
            j = i * NBUF + b

            @pl.when(i > 0)
            def _():
                pltpu.make_async_copy(
                    rows[b], out_hbm.at[pl.ds(base, CHUNK)], sw[b]
                ).wait()

            pltpu.async_copy(w_hbm.at[idx_v.at[j]], rows[b], sg[b]).wait()
            pltpu.async_copy(
                rows[b], out_hbm.at[pl.ds(base + j * CHUNK, CHUNK)], sw[b]
            )
        return carry

    lax.fori_loop(0, CPW // NBUF, body, 0)
    for b in range(NBUF):
        pltpu.make_async_copy(
            rows[b], out_hbm.at[pl.ds(base, CHUNK)], sw[b]
        ).wait()


_TR_BLOCK = 2048


def _transpose_block(wt_ref, wp_ref):
    wp_ref[...] = jnp.transpose(wt_ref[...], (1, 0))


_transpose = pl.pallas_call(
    _transpose_block,
    grid=(pl.cdiv(NUM_E, _TR_BLOCK),),
    in_specs=[pl.BlockSpec((DP, _TR_BLOCK), lambda i: (0, i))],
    out_specs=pl.BlockSpec((_TR_BLOCK, DP), lambda i: (i, 0)),
    out_shape=jax.ShapeDtypeStruct((NUM_E, DP), jnp.float32),
)


def kernel(x, weight):
    x = x.astype(jnp.int32)
    xp = jnp.pad(x, ((0, 0), (0, HP - HIST)), mode="edge")
    xr = xp.reshape(NW, CPW, CHUNK)
    wp = _transpose(lax.transpose(weight, (1, 0)))
    out = _gather(xr, wp)
    return out.reshape(BATCH, HP, DP)[:, :HIST, :D]
